# baseline (speedup 1.0000x reference)
import jax
import jax.numpy as jnp
from jax.experimental import pallas as pl

NUM_Q = 4
K = 1024
D = 256
BETA = 0.25

BM = 1152


def _level_body(res_ref, rown_ref, cb_ref, cbn_ref, cbh_ref, cbm_ref,
                cbl_ref, resout_ref, zq_ref, idx_ref, loss_ref):
    r = res_ref[...]
    rown = rown_ref[...]
    cb = cb_ref[...]
    cbn = cbn_ref[...]
    mm2 = jax.lax.dot_general(
        r + r, cb, (((1,), (1,)), ((), ())),
        preferred_element_type=jnp.float32)
    dist = (rown + cbn) - mm2
    lane = jax.lax.broadcasted_iota(jnp.int32, (BM, K), 1)
    minval = jnp.min(dist, axis=1, keepdims=True)
    at_min = dist - minval <= 5e-6
    idx = jnp.min(jnp.where(at_min, lane, K), axis=1)
    onehot = (lane == idx[:, None]).astype(jnp.bfloat16)
    dn = (((1,), (0,)), ((), ()))
    g_h = jax.lax.dot_general(onehot, cbh_ref[...], dn,
                              preferred_element_type=jnp.float32)
    g_m = jax.lax.dot_general(onehot, cbm_ref[...], dn,
                              preferred_element_type=jnp.float32)
    g_l = jax.lax.dot_general(onehot, cbl_ref[...], dn,
                              preferred_element_type=jnp.float32)
    zq_l = (g_h + g_m) + g_l
    r_new = r - zq_l
    resout_ref[...] = r_new
    zq_ref[...] = zq_l
    idx_ref[...] = idx[:, None]
    diff = zq_l - r_new
    loss = jnp.sum(diff * diff)

    @pl.when(pl.program_id(0) == 0)
    def _():
        loss_ref[...] = jnp.zeros((1, 1), jnp.float32)

    loss_ref[...] = loss_ref[...] + loss


def _level_call(res, rown, cb, cbn, cbh, cbm, cbl):
    M = res.shape[0]
    return pl.pallas_call(
        _level_body,
        grid=(M // BM,),
        in_specs=[
            pl.BlockSpec((BM, D), lambda i: (i, 0)),
            pl.BlockSpec((BM, 1), lambda i: (i, 0)),
            pl.BlockSpec((K, D), lambda i: (0, 0)),
            pl.BlockSpec((1, K), lambda i: (0, 0)),
            pl.BlockSpec((K, D), lambda i: (0, 0)),
            pl.BlockSpec((K, D), lambda i: (0, 0)),
            pl.BlockSpec((K, D), lambda i: (0, 0)),
        ],
        out_specs=[
            pl.BlockSpec((BM, D), lambda i: (i, 0)),
            pl.BlockSpec((BM, D), lambda i: (i, 0)),
            pl.BlockSpec((BM, 1), lambda i: (i, 0)),
            pl.BlockSpec((1, 1), lambda i: (0, 0)),
        ],
        out_shape=[
            jax.ShapeDtypeStruct((M, D), jnp.float32),
            jax.ShapeDtypeStruct((M, D), jnp.float32),
            jax.ShapeDtypeStruct((M, 1), jnp.int32),
            jax.ShapeDtypeStruct((1, 1), jnp.float32),
        ],
    )(res, rown, cb, cbn, cbh, cbm, cbl)


def kernel(z, codebooks):
    B, L, _ = z.shape
    M = B * L
    res = z.reshape(M, D)
    zq_sum = jnp.zeros_like(res)
    total_loss = jnp.float32(0.0)
    idx_cols = []
    for i in range(NUM_Q):
        cb = codebooks[i]
        cbn = jnp.sum(cb ** 2, axis=1)[None, :]
        hi_f32 = jax.lax.bitcast_convert_type(
            jax.lax.bitcast_convert_type(cb, jnp.uint32) & jnp.uint32(0xFFFF0000),
            jnp.float32)
        rem = cb - hi_f32
        mid_f32 = jax.lax.bitcast_convert_type(
            jax.lax.bitcast_convert_type(rem, jnp.uint32) & jnp.uint32(0xFFFF0000),
            jnp.float32)
        low_f32 = rem - mid_f32
        cbh = hi_f32.astype(jnp.bfloat16)
        cbm = mid_f32.astype(jnp.bfloat16)
        cbl = low_f32.astype(jnp.bfloat16)
        rown = jnp.sum(res ** 2, axis=1, keepdims=True)
        res, zq_l, idx, loss = _level_call(res, rown, cb, cbn, cbh, cbm, cbl)
        zq_sum = zq_sum + zq_l
        total_loss = total_loss + loss[0, 0]
        idx_cols.append(idx)
    zq = zq_sum.reshape(B, L, D)
    z_q = z + (zq - z)
    all_indices = (jnp.concatenate(idx_cols, axis=1)
                   .reshape(B, L, NUM_Q).astype(jnp.int64))
    total_loss = (total_loss * ((1.0 + BETA) / (M * D))).astype(jnp.float32)
    return (z_q, all_indices, total_loss)
